# Initial kernel scaffold; baseline (speedup 1.0000x reference)
import jax
import jax.numpy as jnp
from jax.experimental import pallas as pl


def kernel(x1, x2, ln1_g, ln1_b, W_rep, b_rep, ln2_g, ln2_b, attn_w):
    raise NotImplementedError("write your pallas kernel here")


# speedup vs baseline: 33.6931x; 33.6931x over previous
import functools

import jax
import jax.numpy as jnp
from jax.experimental import pallas as pl
from jax.experimental.pallas import tpu as pltpu


def _sortable_i32(x):
    i = jax.lax.bitcast_convert_type(x, jnp.int32)
    return jnp.where(i < 0, i ^ jnp.int32(0x7FFFFFFF), i)


def _mgcc_kernel(ks, N, D, x1_ref, x2_ref, g1_ref, b1_ref, wrep_ref,
                 brep_ref, g2_ref, b2_ref, aw_ref, out_ref):
    f32 = jnp.float32
    x1 = x1_ref[...]
    x2 = x2_ref[...]
    g1 = g1_ref[...]
    b1 = b1_ref[...]

    def ln(x, g, b):
        mu = jnp.mean(x, axis=-1, keepdims=True)
        var = jnp.mean((x - mu) ** 2, axis=-1, keepdims=True)
        return (x - mu) * jax.lax.rsqrt(var + 1e-5) * g + b

    n1 = ln(x1, g1, b1)
    n2 = ln(x2, g1, b1)

    km = jnp.max(n2, axis=0, keepdims=True)
    ke = jnp.exp(n2 - km)
    key_t = ke / jnp.sum(ke, axis=0, keepdims=True)
    qm = jnp.max(n2, axis=1, keepdims=True)
    qe = jnp.exp(n2 - qm)
    qs = qe / jnp.sum(qe, axis=1, keepdims=True)

    context = jax.lax.dot_general(
        key_t, n1, (((0,), (0,)), ((), ())),
        preferred_element_type=f32)

    ikey = _sortable_i32(context)
    ikey3 = jnp.broadcast_to(ikey[None], (4, D, D))
    kidx = jax.lax.broadcasted_iota(jnp.int32, (4, 1, 1), 0)
    kvec = jnp.where(kidx == 0, ks[0],
            jnp.where(kidx == 1, ks[1],
             jnp.where(kidx == 2, ks[2], ks[3]))).astype(jnp.int32)

    lo0 = jnp.full((4, D, 1), jnp.int32(-2**31))
    hi0 = jnp.full((4, D, 1), jnp.int32(2**31 - 1))

    def body(_, carry):
        lo, hi = carry
        mid = (lo & hi) + ((lo ^ hi) >> 1)
        cnt = jnp.sum((ikey3 > mid).astype(jnp.int32), axis=2,
                      keepdims=True)
        pred = cnt >= kvec
        return jnp.where(pred, mid + 1, lo), jnp.where(pred, hi, mid)

    lo, hi = jax.lax.fori_loop(0, 32, body, (lo0, hi0))
    thr = lo

    m = jnp.max(context, axis=1, keepdims=True)
    ec = jnp.exp(context - m)
    mask3 = (ikey3 >= thr).astype(f32)
    S = jnp.sum(ec[None] * mask3, axis=2, keepdims=True)
    aw3 = jnp.where(kidx == 0, aw_ref[0],
           jnp.where(kidx == 1, aw_ref[1],
            jnp.where(kidx == 2, aw_ref[2], aw_ref[3])))
    coef = aw3.astype(f32) / S
    wt = jnp.sum(mask3 * coef, axis=0)
    attn = ec * wt

    attended = jax.lax.dot_general(
        attn, qs, (((1,), (1,)), ((), ())),
        preferred_element_type=f32)

    rep = jnp.dot(wrep_ref[...], attended,
                  preferred_element_type=f32) + brep_ref[...]
    mu = jnp.mean(rep, axis=0, keepdims=True)
    var = jnp.mean((rep - mu) ** 2, axis=0, keepdims=True)
    out = (rep - mu) * jax.lax.rsqrt(var + 1e-5) * g2_ref[...] + b2_ref[...]
    out_ref[...] = out


def kernel(x1, x2, ln1_g, ln1_b, W_rep, b_rep, ln2_g, ln2_b, attn_w):
    B_, H_, W_, C_ = x1.shape
    N = H_ * W_
    D = C_
    ks = (int(D * 1 / 2), int(D * 2 / 3), int(D * 3 / 4), int(D * 4 / 5))

    x1f = x1.reshape(B_ * N, C_)
    x2f = x2.reshape(B_ * N, C_)
    g1 = ln1_g.reshape(1, C_)
    b1 = ln1_b.reshape(1, C_)
    brep = b_rep.reshape(2 * D, 1)
    g2 = ln2_g.reshape(2 * D, 1)
    b2 = ln2_b.reshape(2 * D, 1)

    out = pl.pallas_call(
        functools.partial(_mgcc_kernel, ks, N, D),
        grid=(B_,),
        in_specs=[
            pl.BlockSpec((N, C_), lambda b: (b, 0)),
            pl.BlockSpec((N, C_), lambda b: (b, 0)),
            pl.BlockSpec((1, C_), lambda b: (0, 0)),
            pl.BlockSpec((1, C_), lambda b: (0, 0)),
            pl.BlockSpec((2 * D, D), lambda b: (0, 0)),
            pl.BlockSpec((2 * D, 1), lambda b: (0, 0)),
            pl.BlockSpec((2 * D, 1), lambda b: (0, 0)),
            pl.BlockSpec((2 * D, 1), lambda b: (0, 0)),
            pl.BlockSpec(memory_space=pltpu.SMEM),
        ],
        out_specs=pl.BlockSpec((2 * D, N), lambda b: (b, 0)),
        out_shape=jax.ShapeDtypeStruct((B_ * 2 * D, N), jnp.float32),
    )(x1f, x2f, g1, b1, W_rep, brep, g2, b2, attn_w)

    return out.reshape(B_, 2 * D, H_, W_)


# speedup vs baseline: 42.8067x; 1.2705x over previous
import functools

import jax
import jax.numpy as jnp
from jax.experimental import pallas as pl
from jax.experimental.pallas import tpu as pltpu


def _sortable_i32(x):
    i = jax.lax.bitcast_convert_type(x, jnp.int32)
    return jnp.where(i < 0, i ^ jnp.int32(0x7FFFFFFF), i)


def _mgcc_kernel(ks, N, D, x1_ref, x2_ref, g1_ref, b1_ref, wrep_ref,
                 brep_ref, g2_ref, b2_ref, aw_ref, out_ref):
    f32 = jnp.float32
    x1 = x1_ref[...]
    x2 = x2_ref[...]
    g1 = g1_ref[...]
    b1 = b1_ref[...]

    def ln(x, g, b):
        mu = jnp.mean(x, axis=-1, keepdims=True)
        var = jnp.mean((x - mu) ** 2, axis=-1, keepdims=True)
        return (x - mu) * jax.lax.rsqrt(var + 1e-5) * g + b

    n1 = ln(x1, g1, b1)
    n2 = ln(x2, g1, b1)

    km = jnp.max(n2, axis=0, keepdims=True)
    ke = jnp.exp(n2 - km)
    key_t = ke / jnp.sum(ke, axis=0, keepdims=True)
    qm = jnp.max(n2, axis=1, keepdims=True)
    qe = jnp.exp(n2 - qm)
    qs = qe / jnp.sum(qe, axis=1, keepdims=True)

    context = jax.lax.dot_general(
        key_t, n1, (((0,), (0,)), ((), ())),
        preferred_element_type=f32)

    ikey = _sortable_i32(context)
    ikey3 = jnp.broadcast_to(ikey[None], (4, D, D))
    kidx = jax.lax.broadcasted_iota(jnp.int32, (4, 1, 1), 0)
    kvec = jnp.where(kidx == 0, ks[0],
            jnp.where(kidx == 1, ks[1],
             jnp.where(kidx == 2, ks[2], ks[3]))).astype(jnp.int32)

    lo0 = jnp.full((4, D, 1), jnp.int32(-2**31))
    hi0 = jnp.full((4, D, 1), jnp.int32(2**31 - 1))

    lo, hi = lo0, hi0
    for _ in range(32):
        mid = (lo & hi) + ((lo ^ hi) >> 1)
        cnt = jnp.sum((ikey3 > mid).astype(jnp.int32), axis=2,
                      keepdims=True)
        pred = cnt >= kvec
        lo = jnp.where(pred, mid + 1, lo)
        hi = jnp.where(pred, hi, mid)
    thr = lo

    m = jnp.max(context, axis=1, keepdims=True)
    ec = jnp.exp(context - m)
    mask3 = (ikey3 >= thr).astype(f32)
    S = jnp.sum(ec[None] * mask3, axis=2, keepdims=True)
    aw3 = jnp.where(kidx == 0, aw_ref[0],
           jnp.where(kidx == 1, aw_ref[1],
            jnp.where(kidx == 2, aw_ref[2], aw_ref[3])))
    coef = aw3.astype(f32) / S
    wt = jnp.sum(mask3 * coef, axis=0)
    attn = ec * wt

    attended = jax.lax.dot_general(
        attn, qs, (((1,), (1,)), ((), ())),
        preferred_element_type=f32)

    rep = jnp.dot(wrep_ref[...], attended,
                  preferred_element_type=f32) + brep_ref[...]
    mu = jnp.mean(rep, axis=0, keepdims=True)
    var = jnp.mean((rep - mu) ** 2, axis=0, keepdims=True)
    out = (rep - mu) * jax.lax.rsqrt(var + 1e-5) * g2_ref[...] + b2_ref[...]
    out_ref[...] = out


def kernel(x1, x2, ln1_g, ln1_b, W_rep, b_rep, ln2_g, ln2_b, attn_w):
    B_, H_, W_, C_ = x1.shape
    N = H_ * W_
    D = C_
    ks = (int(D * 1 / 2), int(D * 2 / 3), int(D * 3 / 4), int(D * 4 / 5))

    x1f = x1.reshape(B_ * N, C_)
    x2f = x2.reshape(B_ * N, C_)
    g1 = ln1_g.reshape(1, C_)
    b1 = ln1_b.reshape(1, C_)
    brep = b_rep.reshape(2 * D, 1)
    g2 = ln2_g.reshape(2 * D, 1)
    b2 = ln2_b.reshape(2 * D, 1)

    out = pl.pallas_call(
        functools.partial(_mgcc_kernel, ks, N, D),
        grid=(B_,),
        in_specs=[
            pl.BlockSpec((N, C_), lambda b: (b, 0)),
            pl.BlockSpec((N, C_), lambda b: (b, 0)),
            pl.BlockSpec((1, C_), lambda b: (0, 0)),
            pl.BlockSpec((1, C_), lambda b: (0, 0)),
            pl.BlockSpec((2 * D, D), lambda b: (0, 0)),
            pl.BlockSpec((2 * D, 1), lambda b: (0, 0)),
            pl.BlockSpec((2 * D, 1), lambda b: (0, 0)),
            pl.BlockSpec((2 * D, 1), lambda b: (0, 0)),
            pl.BlockSpec(memory_space=pltpu.SMEM),
        ],
        out_specs=pl.BlockSpec((2 * D, N), lambda b: (b, 0)),
        out_shape=jax.ShapeDtypeStruct((B_ * 2 * D, N), jnp.float32),
    )(x1f, x2f, g1, b1, W_rep, brep, g2, b2, attn_w)

    return out.reshape(B_, 2 * D, H_, W_)


# speedup vs baseline: 50.9982x; 1.1914x over previous
import functools

import jax
import jax.numpy as jnp
from jax.experimental import pallas as pl
from jax.experimental.pallas import tpu as pltpu


def _sortable_i32(x):
    i = jax.lax.bitcast_convert_type(x, jnp.int32)
    return jnp.where(i < 0, i ^ jnp.int32(0x7FFFFFFF), i)


def _mgcc_kernel(ks, N, D, x1_ref, x2_ref, g1_ref, b1_ref, wrep_ref,
                 brep_ref, g2_ref, b2_ref, aw_ref, out_ref):
    f32 = jnp.float32
    x1 = x1_ref[...]
    x2 = x2_ref[...]
    g1 = g1_ref[...]
    b1 = b1_ref[...]

    def ln(x, g, b):
        mu = jnp.mean(x, axis=-1, keepdims=True)
        var = jnp.mean((x - mu) ** 2, axis=-1, keepdims=True)
        return (x - mu) * jax.lax.rsqrt(var + 1e-5) * g + b

    n1 = ln(x1, g1, b1)
    n2 = ln(x2, g1, b1)

    km = jnp.max(n2, axis=0, keepdims=True)
    ke = jnp.exp(n2 - km)
    key_t = ke / jnp.sum(ke, axis=0, keepdims=True)
    qm = jnp.max(n2, axis=1, keepdims=True)
    qe = jnp.exp(n2 - qm)
    qs = qe / jnp.sum(qe, axis=1, keepdims=True)

    context = jax.lax.dot_general(
        key_t, n1, (((0,), (0,)), ((), ())),
        preferred_element_type=f32)

    ikey = _sortable_i32(context)
    ikey3 = jnp.broadcast_to(ikey[None], (4, D, D))
    kidx = jax.lax.broadcasted_iota(jnp.int32, (4, 1, 1), 0)
    kvec = jnp.where(kidx == 0, ks[0],
            jnp.where(kidx == 1, ks[1],
             jnp.where(kidx == 2, ks[2], ks[3]))).astype(jnp.int32)

    lo = jnp.full((4, D, 1), jnp.int32(-2**31))
    hi = jnp.full((4, D, 1), jnp.int32(2**31 - 1))
    kvecf = kvec.astype(f32)
    ones_v = jnp.ones((D, 1), jnp.bfloat16)
    for _ in range(32):
        mid = (lo & hi) + ((lo ^ hi) >> 1)
        mask = (ikey3 > mid).astype(jnp.bfloat16).reshape(4 * D, D)
        cnt = jnp.dot(mask, ones_v,
                      preferred_element_type=f32).reshape(4, D, 1)
        pred = cnt >= kvecf
        lo = jnp.where(pred, mid + 1, lo)
        hi = jnp.where(pred, hi, mid)
    thr = lo

    m = jnp.max(context, axis=1, keepdims=True)
    ec = jnp.exp(context - m)
    mask3 = (ikey3 >= thr).astype(f32)
    S = jnp.sum(ec[None] * mask3, axis=2, keepdims=True)
    aw3 = jnp.where(kidx == 0, aw_ref[0],
           jnp.where(kidx == 1, aw_ref[1],
            jnp.where(kidx == 2, aw_ref[2], aw_ref[3])))
    coef = aw3.astype(f32) / S
    wt = jnp.sum(mask3 * coef, axis=0)
    attn = ec * wt

    attended = jax.lax.dot_general(
        attn, qs, (((1,), (1,)), ((), ())),
        preferred_element_type=f32)

    rep = jnp.dot(wrep_ref[...], attended,
                  preferred_element_type=f32) + brep_ref[...]
    mu = jnp.mean(rep, axis=0, keepdims=True)
    var = jnp.mean((rep - mu) ** 2, axis=0, keepdims=True)
    out = (rep - mu) * jax.lax.rsqrt(var + 1e-5) * g2_ref[...] + b2_ref[...]
    out_ref[...] = out


def kernel(x1, x2, ln1_g, ln1_b, W_rep, b_rep, ln2_g, ln2_b, attn_w):
    B_, H_, W_, C_ = x1.shape
    N = H_ * W_
    D = C_
    ks = (int(D * 1 / 2), int(D * 2 / 3), int(D * 3 / 4), int(D * 4 / 5))

    x1f = x1.reshape(B_ * N, C_)
    x2f = x2.reshape(B_ * N, C_)
    g1 = ln1_g.reshape(1, C_)
    b1 = ln1_b.reshape(1, C_)
    brep = b_rep.reshape(2 * D, 1)
    g2 = ln2_g.reshape(2 * D, 1)
    b2 = ln2_b.reshape(2 * D, 1)

    out = pl.pallas_call(
        functools.partial(_mgcc_kernel, ks, N, D),
        grid=(B_,),
        in_specs=[
            pl.BlockSpec((N, C_), lambda b: (b, 0)),
            pl.BlockSpec((N, C_), lambda b: (b, 0)),
            pl.BlockSpec((1, C_), lambda b: (0, 0)),
            pl.BlockSpec((1, C_), lambda b: (0, 0)),
            pl.BlockSpec((2 * D, D), lambda b: (0, 0)),
            pl.BlockSpec((2 * D, 1), lambda b: (0, 0)),
            pl.BlockSpec((2 * D, 1), lambda b: (0, 0)),
            pl.BlockSpec((2 * D, 1), lambda b: (0, 0)),
            pl.BlockSpec(memory_space=pltpu.SMEM),
        ],
        out_specs=pl.BlockSpec((2 * D, N), lambda b: (b, 0)),
        out_shape=jax.ShapeDtypeStruct((B_ * 2 * D, N), jnp.float32),
    )(x1f, x2f, g1, b1, W_rep, brep, g2, b2, attn_w)

    return out.reshape(B_, 2 * D, H_, W_)
